# baseline (speedup 1.0000x reference)
import functools

import jax
import jax.numpy as jnp
from jax import lax
from jax.experimental import pallas as pl
from jax.experimental.pallas import tpu as pltpu
from jax.experimental.pallas import tpu_sc as plsc

_NC = 2
_NS = 16
_NW = _NC * _NS
_C = 80


def _sc_segment_sum(h, edge_index, zeros):
    n, feat = h.shape
    dt = h.dtype
    e = edge_index.shape[1]
    ew = e // _NW
    c = _C
    nchunk = ew // c
    rpt = zeros.shape[0]
    n_pad = rpt * _NS

    mesh = plsc.VectorSubcoreMesh(core_axis_name="c", subcore_axis_name="s")

    k = 5
    ngroups = nchunk // k
    npairs = ngroups // 2

    @functools.partial(
        pl.kernel,
        out_type=jax.ShapeDtypeStruct((_NC, n_pad, feat), dt),
        mesh=mesh,
        scratch_types=[
            pltpu.VMEM((ew,), jnp.int32),
            pltpu.VMEM((ew,), jnp.int32),
            pltpu.VMEM((k, c, feat), dt),
            pltpu.VMEM((k, c, feat), dt),
            pltpu.VMEM_SHARED((n_pad, feat), dt),
            pltpu.SemaphoreType.DMA,
            pltpu.SemaphoreType.DMA,
            pltpu.SemaphoreType.DMA,
            pltpu.SemaphoreType.DMA,
        ],
        compiler_params=pltpu.CompilerParams(use_tc_tiling_on_sc=False),
    )
    def sc_kernel(e_hbm, h_hbm, z_hbm, out_hbm,
                  src_v, dst_v, rows_a, rows_b, agg_sh,
                  gsem_a, gsem_b, ssem_a, ssem_b):
        cid = lax.axis_index("c")
        sid = lax.axis_index("s")
        wid = sid * _NC + cid
        pltpu.sync_copy(z_hbm, agg_sh.at[pl.ds(sid * rpt, rpt)])
        pltpu.sync_copy(e_hbm.at[0, pl.ds(wid * ew, ew)], src_v)
        pltpu.sync_copy(e_hbm.at[1, pl.ds(wid * ew, ew)], dst_v)
        plsc.subcore_barrier()

        def fire_gathers(jbase, rows, sem):
            for b in range(k):
                pltpu.async_copy(
                    h_hbm.at[src_v.at[pl.ds((jbase + b) * c, c)]],
                    rows.at[b], sem)

        def drain(rows, sem):
            for b in range(k):
                pltpu.make_async_copy(z_hbm.at[pl.ds(0, c)],
                                      rows.at[b], sem).wait()

        def fire_scatters(jbase, rows, sem):
            for b in range(k):
                pltpu.async_copy(
                    rows.at[b],
                    agg_sh.at[dst_v.at[pl.ds((jbase + b) * c, c)]],
                    sem, add=True)

        fire_gathers(0, rows_a, gsem_a)

        def pair(t, carry):
            g0 = 2 * t * k
            g1 = g0 + k
            drain(rows_a, gsem_a)
            fire_gathers(g1, rows_b, gsem_b)
            fire_scatters(g0, rows_a, ssem_a)
            drain(rows_a, ssem_a)

            @pl.when(t + 1 < npairs)
            def _():
                fire_gathers(g0 + 2 * k, rows_a, gsem_a)

            if ngroups % 2 == 1:
                @pl.when(t + 1 == npairs)
                def _():
                    fire_gathers((2 * npairs) * k, rows_a, gsem_a)

            drain(rows_b, gsem_b)
            fire_scatters(g1, rows_b, ssem_b)
            drain(rows_b, ssem_b)
            return carry

        lax.fori_loop(0, npairs, pair, 0)

        if ngroups % 2 == 1:
            gt = (2 * npairs) * k
            drain(rows_a, gsem_a)
            fire_scatters(gt, rows_a, ssem_a)
            drain(rows_a, ssem_a)

        plsc.subcore_barrier()
        pltpu.sync_copy(agg_sh.at[pl.ds(sid * rpt, rpt)],
                        out_hbm.at[cid, pl.ds(sid * rpt, rpt)])

    return sc_kernel(edge_index, h, zeros)


def _dense_in(x_ref, w_ref, wr_ref, br_ref, h_ref, r_ref):
    x = x_ref[...].astype(jnp.bfloat16)
    h_ref[...] = jnp.dot(
        x, w_ref[...].astype(jnp.bfloat16),
        preferred_element_type=jnp.float32).astype(h_ref.dtype)
    r_ref[...] = jnp.maximum(
        jnp.dot(x, wr_ref[...].astype(jnp.bfloat16),
                preferred_element_type=jnp.float32)
        + br_ref[...], 0.0)


def _combine_next(agg_ref, r_ref, b_ref, g_ref, be_ref,
                  w_ref, wr_ref, br_ref, h_ref, r2_ref):
    inv = 1.0 / jnp.sqrt(jnp.float32(1.0 + 1e-5))
    agg = (agg_ref[0].astype(jnp.float32)
           + agg_ref[1].astype(jnp.float32))
    x = ((g_ref[...] * inv)
         * (jnp.maximum(agg + b_ref[...], 0.0) + r_ref[...]) + be_ref[...])
    xb = x.astype(jnp.bfloat16)
    h_ref[...] = jnp.dot(
        xb, w_ref[...].astype(jnp.bfloat16),
        preferred_element_type=jnp.float32).astype(h_ref.dtype)
    r2_ref[...] = jnp.maximum(
        jnp.dot(xb, wr_ref[...].astype(jnp.bfloat16),
                preferred_element_type=jnp.float32)
        + br_ref[...], 0.0)


def _combine_out(agg_ref, r_ref, b_ref, g_ref, be_ref, o_ref):
    inv = 1.0 / jnp.sqrt(jnp.float32(1.0 + 1e-5))
    agg = (agg_ref[0].astype(jnp.float32)
           + agg_ref[1].astype(jnp.float32))
    o_ref[...] = ((g_ref[...] * inv)
                  * (jnp.maximum(agg + b_ref[...], 0.0) + r_ref[...])
                  + be_ref[...])


def kernel(feats, edge_index, W1, b1, Wr1, br1, g1, be1,
           W2, b2, Wr2, br2, g2, be2):
    n, d_in = feats.shape
    h = W1.shape[1]
    rpt = ((n + 8 * _NS - 1) // (8 * _NS)) * 8
    sc_dt = jnp.bfloat16
    zeros = jnp.zeros((rpt, h), sc_dt)

    f32 = jnp.float32
    b1r, br1r, g1r, be1r = (v.reshape(1, h) for v in (b1, br1, g1, be1))
    b2r, br2r, g2r, be2r = (v.reshape(1, h) for v in (b2, br2, g2, be2))

    grid = 10
    br = n // grid
    n_pad = rpt * _NS
    row_spec = pl.BlockSpec((br, h), lambda i: (i, 0))
    full64 = pl.BlockSpec((1, h), lambda i: (0, 0))
    agg_spec = pl.BlockSpec((_NC, br, h), lambda i: (0, i, 0))

    h1, r1 = pl.pallas_call(
        _dense_in,
        grid=(grid,),
        in_specs=[pl.BlockSpec((br, d_in), lambda i: (i, 0)),
                  pl.BlockSpec((d_in, h), lambda i: (0, 0)),
                  pl.BlockSpec((d_in, h), lambda i: (0, 0)),
                  full64],
        out_specs=[row_spec, row_spec],
        out_shape=[jax.ShapeDtypeStruct((n, h), sc_dt),
                   jax.ShapeDtypeStruct((n, h), f32)],
    )(feats, W1, Wr1, br1r)

    agg1 = _sc_segment_sum(h1, edge_index, zeros)

    wspec = pl.BlockSpec((h, h), lambda i: (0, 0))
    h2, r2 = pl.pallas_call(
        _combine_next,
        grid=(grid,),
        in_specs=[agg_spec, row_spec, full64, full64, full64,
                  wspec, wspec, full64],
        out_specs=[row_spec, row_spec],
        out_shape=[jax.ShapeDtypeStruct((n, h), sc_dt),
                   jax.ShapeDtypeStruct((n, h), f32)],
    )(agg1, r1, b1r, g1r, be1r, W2, Wr2, br2r)

    agg2 = _sc_segment_sum(h2, edge_index, zeros)

    out = pl.pallas_call(
        _combine_out,
        grid=(grid,),
        in_specs=[agg_spec, row_spec, full64, full64, full64],
        out_specs=row_spec,
        out_shape=jax.ShapeDtypeStruct((n, h), f32),
    )(agg2, r2, b2r, g2r, be2r)

    return out


# speedup vs baseline: 1.0572x; 1.0572x over previous
import functools

import jax
import jax.numpy as jnp
from jax import lax
from jax.experimental import pallas as pl
from jax.experimental.pallas import tpu as pltpu
from jax.experimental.pallas import tpu_sc as plsc

_NC = 2
_NS = 16
_NW = _NC * _NS
_C = 80


def _sc_segment_sum(h, edge_index, zeros):
    n, feat = h.shape
    dt = h.dtype
    e = edge_index.shape[1]
    ew = e // _NW
    c = _C
    nchunk = ew // c
    rpt = zeros.shape[0]
    n_pad = rpt * _NS

    mesh = plsc.VectorSubcoreMesh(core_axis_name="c", subcore_axis_name="s")

    k = 5
    ngroups = nchunk // k
    npairs = ngroups // 2

    @functools.partial(
        pl.kernel,
        out_type=jax.ShapeDtypeStruct((_NC, n_pad, feat), dt),
        mesh=mesh,
        scratch_types=[
            pltpu.VMEM((ew,), jnp.int32),
            pltpu.VMEM((ew,), jnp.int32),
            pltpu.VMEM((k, c, feat), dt),
            pltpu.VMEM((k, c, feat), dt),
            pltpu.VMEM_SHARED((n_pad, feat), dt),
            pltpu.SemaphoreType.DMA,
            pltpu.SemaphoreType.DMA,
            pltpu.SemaphoreType.DMA,
            pltpu.SemaphoreType.DMA,
        ],
        compiler_params=pltpu.CompilerParams(use_tc_tiling_on_sc=False),
    )
    def sc_kernel(e_hbm, h_hbm, z_hbm, out_hbm,
                  src_v, dst_v, rows_a, rows_b, agg_sh,
                  gsem_a, gsem_b, ssem_a, ssem_b):
        cid = lax.axis_index("c")
        sid = lax.axis_index("s")
        wid = sid * _NC + cid
        pltpu.sync_copy(e_hbm.at[0, pl.ds(wid * ew, ew)], src_v)

        def fire_gathers(jbase, rows, sem):
            for b in range(k):
                pltpu.async_copy(
                    h_hbm.at[src_v.at[pl.ds((jbase + b) * c, c)]],
                    rows.at[b], sem)

        def drain(rows, sem):
            for b in range(k):
                pltpu.make_async_copy(z_hbm.at[pl.ds(0, c)],
                                      rows.at[b], sem).wait()

        def fire_scatters(jbase, rows, sem):
            for b in range(k):
                pltpu.async_copy(
                    rows.at[b],
                    agg_sh.at[dst_v.at[pl.ds((jbase + b) * c, c)]],
                    sem, add=True)

        fire_gathers(0, rows_a, gsem_a)
        pltpu.sync_copy(z_hbm, agg_sh.at[pl.ds(sid * rpt, rpt)])
        pltpu.sync_copy(e_hbm.at[1, pl.ds(wid * ew, ew)], dst_v)
        plsc.subcore_barrier()

        def pair(t, carry):
            g0 = 2 * t * k
            g1 = g0 + k
            drain(rows_a, gsem_a)
            fire_gathers(g1, rows_b, gsem_b)
            fire_scatters(g0, rows_a, ssem_a)
            drain(rows_a, ssem_a)

            @pl.when(t + 1 < npairs)
            def _():
                fire_gathers(g0 + 2 * k, rows_a, gsem_a)

            if ngroups % 2 == 1:
                @pl.when(t + 1 == npairs)
                def _():
                    fire_gathers((2 * npairs) * k, rows_a, gsem_a)

            drain(rows_b, gsem_b)
            fire_scatters(g1, rows_b, ssem_b)
            drain(rows_b, ssem_b)
            return carry

        lax.fori_loop(0, npairs, pair, 0)

        if ngroups % 2 == 1:
            gt = (2 * npairs) * k
            drain(rows_a, gsem_a)
            fire_scatters(gt, rows_a, ssem_a)
            drain(rows_a, ssem_a)

        plsc.subcore_barrier()
        pltpu.sync_copy(agg_sh.at[pl.ds(sid * rpt, rpt)],
                        out_hbm.at[cid, pl.ds(sid * rpt, rpt)])

    return sc_kernel(edge_index, h, zeros)


def _dense_in(x_ref, w_ref, wr_ref, br_ref, h_ref, r_ref):
    x = x_ref[...].astype(jnp.bfloat16)
    h_ref[...] = jnp.dot(
        x, w_ref[...].astype(jnp.bfloat16),
        preferred_element_type=jnp.float32).astype(h_ref.dtype)
    r_ref[...] = jnp.maximum(
        jnp.dot(x, wr_ref[...].astype(jnp.bfloat16),
                preferred_element_type=jnp.float32)
        + br_ref[...], 0.0)


def _combine_next(agg_ref, r_ref, b_ref, g_ref, be_ref,
                  w_ref, wr_ref, br_ref, h_ref, r2_ref):
    inv = 1.0 / jnp.sqrt(jnp.float32(1.0 + 1e-5))
    nrows = r_ref.shape[0]
    agg = (agg_ref[0, :nrows, :].astype(jnp.float32)
           + agg_ref[1, :nrows, :].astype(jnp.float32))
    x = ((g_ref[...] * inv)
         * (jnp.maximum(agg + b_ref[...], 0.0) + r_ref[...]) + be_ref[...])
    xb = x.astype(jnp.bfloat16)
    h_ref[...] = jnp.dot(
        xb, w_ref[...].astype(jnp.bfloat16),
        preferred_element_type=jnp.float32).astype(h_ref.dtype)
    r2_ref[...] = jnp.maximum(
        jnp.dot(xb, wr_ref[...].astype(jnp.bfloat16),
                preferred_element_type=jnp.float32)
        + br_ref[...], 0.0)


def _combine_out(agg_ref, r_ref, b_ref, g_ref, be_ref, o_ref):
    inv = 1.0 / jnp.sqrt(jnp.float32(1.0 + 1e-5))
    nrows = r_ref.shape[0]
    agg = (agg_ref[0, :nrows, :].astype(jnp.float32)
           + agg_ref[1, :nrows, :].astype(jnp.float32))
    o_ref[...] = ((g_ref[...] * inv)
                  * (jnp.maximum(agg + b_ref[...], 0.0) + r_ref[...])
                  + be_ref[...])


def kernel(feats, edge_index, W1, b1, Wr1, br1, g1, be1,
           W2, b2, Wr2, br2, g2, be2):
    n, d_in = feats.shape
    h = W1.shape[1]
    rpt = ((n + 8 * _NS - 1) // (8 * _NS)) * 8
    sc_dt = jnp.bfloat16
    zeros = jnp.zeros((rpt, h), sc_dt)

    f32 = jnp.float32
    b1r, br1r, g1r, be1r = (v.reshape(1, h) for v in (b1, br1, g1, be1))
    b2r, br2r, g2r, be2r = (v.reshape(1, h) for v in (b2, br2, g2, be2))

    h1, r1 = pl.pallas_call(
        _dense_in,
        out_shape=[jax.ShapeDtypeStruct((n, h), sc_dt),
                   jax.ShapeDtypeStruct((n, h), f32)],
    )(feats, W1, Wr1, br1r)

    agg1 = _sc_segment_sum(h1, edge_index, zeros)

    h2, r2 = pl.pallas_call(
        _combine_next,
        out_shape=[jax.ShapeDtypeStruct((n, h), sc_dt),
                   jax.ShapeDtypeStruct((n, h), f32)],
    )(agg1, r1, b1r, g1r, be1r, W2, Wr2, br2r)

    agg2 = _sc_segment_sum(h2, edge_index, zeros)

    out = pl.pallas_call(
        _combine_out,
        out_shape=jax.ShapeDtypeStruct((n, h), f32),
    )(agg2, r2, b2r, g2r, be2r)

    return out


# speedup vs baseline: 1.1005x; 1.0409x over previous
import functools

import jax
import jax.numpy as jnp
from jax import lax
from jax.experimental import pallas as pl
from jax.experimental.pallas import tpu as pltpu
from jax.experimental.pallas import tpu_sc as plsc

_NC = 2
_NS = 16
_NW = _NC * _NS
_C = 80


def _sc_segment_sum(h, edge_index, zeros):
    n, feat = h.shape
    dt = h.dtype
    e = edge_index.shape[1]
    ew = e // _NW
    c = _C
    nchunk = ew // c
    rpt = zeros.shape[0]
    n_pad = rpt * _NS

    mesh = plsc.VectorSubcoreMesh(core_axis_name="c", subcore_axis_name="s")

    k = 5
    ngroups = nchunk // k
    npairs = ngroups // 2

    @functools.partial(
        pl.kernel,
        out_type=jax.ShapeDtypeStruct((_NC, n_pad, feat), dt),
        mesh=mesh,
        scratch_types=[
            pltpu.VMEM((ew,), jnp.int32),
            pltpu.VMEM((ew,), jnp.int32),
            pltpu.VMEM((k, c, feat), dt),
            pltpu.VMEM((k, c, feat), dt),
            pltpu.VMEM_SHARED((n_pad, feat), dt),
            pltpu.SemaphoreType.DMA,
            pltpu.SemaphoreType.DMA,
            pltpu.SemaphoreType.DMA,
            pltpu.SemaphoreType.DMA,
        ],
        compiler_params=pltpu.CompilerParams(use_tc_tiling_on_sc=False),
    )
    def sc_kernel(e_hbm, h_hbm, z_hbm, out_hbm,
                  src_v, dst_v, rows_a, rows_b, agg_sh,
                  gsem_a, gsem_b, ssem_a, ssem_b):
        cid = lax.axis_index("c")
        sid = lax.axis_index("s")
        wid = sid * _NC + cid
        pltpu.sync_copy(e_hbm.at[0, pl.ds(wid * ew, ew)], src_v)

        def fire_gathers(jbase, rows, sem):
            for b in range(k):
                pltpu.async_copy(
                    h_hbm.at[src_v.at[pl.ds((jbase + b) * c, c)]],
                    rows.at[b], sem)

        def drain(rows, sem):
            for b in range(k):
                pltpu.make_async_copy(z_hbm.at[pl.ds(0, c)],
                                      rows.at[b], sem).wait()

        def fire_scatters(jbase, rows, sem):
            for b in range(k):
                pltpu.async_copy(
                    rows.at[b],
                    agg_sh.at[dst_v.at[pl.ds((jbase + b) * c, c)]],
                    sem, add=True)

        fire_gathers(0, rows_a, gsem_a)
        pltpu.sync_copy(z_hbm, agg_sh.at[pl.ds(sid * rpt, rpt)])
        pltpu.sync_copy(e_hbm.at[1, pl.ds(wid * ew, ew)], dst_v)
        plsc.subcore_barrier()

        def pair(t, carry):
            g0 = 2 * t * k
            g1 = g0 + k
            drain(rows_a, gsem_a)
            fire_gathers(g1, rows_b, gsem_b)
            fire_scatters(g0, rows_a, ssem_a)
            drain(rows_a, ssem_a)

            @pl.when(t + 1 < npairs)
            def _():
                fire_gathers(g0 + 2 * k, rows_a, gsem_a)

            if ngroups % 2 == 1:
                @pl.when(t + 1 == npairs)
                def _():
                    fire_gathers((2 * npairs) * k, rows_a, gsem_a)

            drain(rows_b, gsem_b)
            fire_scatters(g1, rows_b, ssem_b)
            drain(rows_b, ssem_b)
            return carry

        lax.fori_loop(0, npairs, pair, 0)

        if ngroups % 2 == 1:
            gt = (2 * npairs) * k
            drain(rows_a, gsem_a)
            fire_scatters(gt, rows_a, ssem_a)
            drain(rows_a, ssem_a)

        plsc.subcore_barrier()
        pltpu.sync_copy(agg_sh.at[pl.ds(sid * rpt, rpt)],
                        out_hbm.at[cid, pl.ds(sid * rpt, rpt)])

    return sc_kernel(edge_index, h, zeros)


def _dense_in(x_ref, w_ref, wr_ref, br_ref, h_ref, r_ref):
    x = x_ref[...].astype(jnp.bfloat16)
    h_ref[...] = jnp.dot(
        x, w_ref[...].astype(jnp.bfloat16),
        preferred_element_type=jnp.float32).astype(h_ref.dtype)
    r_ref[...] = jnp.maximum(
        jnp.dot(x, wr_ref[...].astype(jnp.bfloat16),
                preferred_element_type=jnp.float32)
        + br_ref[...], 0.0)


def _combine_next(agg_ref, r_ref, b_ref, g_ref, be_ref,
                  w_ref, wr_ref, br_ref, h_ref, r2_ref):
    inv = 1.0 / jnp.sqrt(jnp.float32(1.0 + 1e-5))
    nrows = r_ref.shape[0]
    agg = (agg_ref[0, :nrows, :].astype(jnp.float32)
           + agg_ref[1, :nrows, :].astype(jnp.float32))
    x = ((g_ref[...] * inv)
         * (jnp.maximum(agg + b_ref[...], 0.0) + r_ref[...]) + be_ref[...])
    xb = x.astype(jnp.bfloat16)
    h_ref[...] = jnp.dot(
        xb, w_ref[...].astype(jnp.bfloat16),
        preferred_element_type=jnp.float32).astype(h_ref.dtype)
    r2_ref[...] = jnp.maximum(
        jnp.dot(xb, wr_ref[...].astype(jnp.bfloat16),
                preferred_element_type=jnp.float32)
        + br_ref[...], 0.0)


def _combine_out(agg_ref, r_ref, b_ref, g_ref, be_ref, o_ref):
    inv = 1.0 / jnp.sqrt(jnp.float32(1.0 + 1e-5))
    nrows = r_ref.shape[0]
    agg = (agg_ref[0, :nrows, :].astype(jnp.float32)
           + agg_ref[1, :nrows, :].astype(jnp.float32))
    out = ((g_ref[...] * inv)
           * (jnp.maximum(agg + b_ref[...], 0.0) + r_ref[...])
           + be_ref[...])
    o_ref[...] = out.T


def kernel(feats, edge_index, W1, b1, Wr1, br1, g1, be1,
           W2, b2, Wr2, br2, g2, be2):
    n, d_in = feats.shape
    h = W1.shape[1]
    rpt = ((n + 8 * _NS - 1) // (8 * _NS)) * 8
    sc_dt = jnp.bfloat16
    zeros = jnp.zeros((rpt, h), sc_dt)

    f32 = jnp.float32
    b1r, br1r, g1r, be1r = (v.reshape(1, h) for v in (b1, br1, g1, be1))
    b2r, br2r, g2r, be2r = (v.reshape(1, h) for v in (b2, br2, g2, be2))

    h1, r1 = pl.pallas_call(
        _dense_in,
        out_shape=[jax.ShapeDtypeStruct((n, h), sc_dt),
                   jax.ShapeDtypeStruct((n, h), f32)],
    )(feats, W1, Wr1, br1r)

    agg1 = _sc_segment_sum(h1, edge_index, zeros)

    h2, r2 = pl.pallas_call(
        _combine_next,
        out_shape=[jax.ShapeDtypeStruct((n, h), sc_dt),
                   jax.ShapeDtypeStruct((n, h), f32)],
    )(agg1, r1, b1r, g1r, be1r, W2, Wr2, br2r)

    agg2 = _sc_segment_sum(h2, edge_index, zeros)

    out_t = pl.pallas_call(
        _combine_out,
        out_shape=jax.ShapeDtypeStruct((h, n), f32),
    )(agg2, r2, b2r, g2r, be2r)

    return out_t.T


# speedup vs baseline: 1.1211x; 1.0187x over previous
import functools

import jax
import jax.numpy as jnp
from jax import lax
from jax.experimental import pallas as pl
from jax.experimental.pallas import tpu as pltpu
from jax.experimental.pallas import tpu_sc as plsc

_NC = 2
_NS = 16
_NW = _NC * _NS
_C = 80


def _sc_segment_sum(h, edge_index, zeros):
    n, feat = h.shape
    dt = h.dtype
    e = edge_index.shape[1]
    ew = e // _NW
    c = _C
    nchunk = ew // c
    rpt = zeros.shape[0]
    n_pad = rpt * _NS

    mesh = plsc.VectorSubcoreMesh(core_axis_name="c", subcore_axis_name="s")

    k = 5
    ngroups = nchunk // k
    npairs = ngroups // 2

    @functools.partial(
        pl.kernel,
        out_type=jax.ShapeDtypeStruct((_NC, n_pad, feat), dt),
        mesh=mesh,
        scratch_types=[
            pltpu.VMEM((ew,), jnp.int32),
            pltpu.VMEM((ew,), jnp.int32),
            pltpu.VMEM((k, c, feat), dt),
            pltpu.VMEM((k, c, feat), dt),
            pltpu.VMEM_SHARED((n_pad, feat), dt),
            pltpu.SemaphoreType.DMA,
            pltpu.SemaphoreType.DMA,
            pltpu.SemaphoreType.DMA,
            pltpu.SemaphoreType.DMA,
        ],
        compiler_params=pltpu.CompilerParams(use_tc_tiling_on_sc=False),
    )
    def sc_kernel(e_hbm, h_hbm, z_hbm, out_hbm,
                  src_v, dst_v, rows_a, rows_b, agg_sh,
                  gsem_a, gsem_b, ssem_a, ssem_b):
        cid = lax.axis_index("c")
        sid = lax.axis_index("s")
        wid = sid * _NC + cid
        pltpu.sync_copy(e_hbm.at[0, pl.ds(wid * ew, ew)], src_v)

        def fire_gathers(jbase, rows, sem):
            for b in range(k):
                pltpu.async_copy(
                    h_hbm.at[src_v.at[pl.ds((jbase + b) * c, c)]],
                    rows.at[b], sem)

        def drain(rows, sem):
            for b in range(k):
                pltpu.make_async_copy(z_hbm.at[pl.ds(0, c)],
                                      rows.at[b], sem).wait()

        def fire_scatters(jbase, rows, sem):
            for b in range(k):
                pltpu.async_copy(
                    rows.at[b],
                    agg_sh.at[dst_v.at[pl.ds((jbase + b) * c, c)]],
                    sem, add=True)

        fire_gathers(0, rows_a, gsem_a)
        pltpu.sync_copy(z_hbm, agg_sh.at[pl.ds(sid * rpt, rpt)])
        pltpu.sync_copy(e_hbm.at[1, pl.ds(wid * ew, ew)], dst_v)
        plsc.subcore_barrier()

        def pair(t, carry):
            g0 = 2 * t * k
            g1 = g0 + k
            drain(rows_a, gsem_a)
            fire_gathers(g1, rows_b, gsem_b)
            fire_scatters(g0, rows_a, ssem_a)
            drain(rows_a, ssem_a)

            @pl.when(t + 1 < npairs)
            def _():
                fire_gathers(g0 + 2 * k, rows_a, gsem_a)

            if ngroups % 2 == 1:
                @pl.when(t + 1 == npairs)
                def _():
                    fire_gathers((2 * npairs) * k, rows_a, gsem_a)

            drain(rows_b, gsem_b)
            fire_scatters(g1, rows_b, ssem_b)
            drain(rows_b, ssem_b)
            return carry

        lax.fori_loop(0, npairs, pair, 0)

        if ngroups % 2 == 1:
            gt = (2 * npairs) * k
            drain(rows_a, gsem_a)
            fire_scatters(gt, rows_a, ssem_a)
            drain(rows_a, ssem_a)

        plsc.subcore_barrier()
        pltpu.sync_copy(agg_sh.at[pl.ds(sid * rpt, rpt)],
                        out_hbm.at[cid, pl.ds(sid * rpt, rpt)])

    return sc_kernel(edge_index, h, zeros)


def _dense_h(x_ref, w_ref, h_ref):
    x = x_ref[...].astype(jnp.bfloat16)
    h_ref[...] = jnp.dot(
        x, w_ref[...].astype(jnp.bfloat16),
        preferred_element_type=jnp.float32).astype(h_ref.dtype)


def _dense_res(x_ref, wr_ref, br_ref, r_ref):
    x = x_ref[...].astype(jnp.bfloat16)
    r_ref[...] = jnp.maximum(
        jnp.dot(x, wr_ref[...].astype(jnp.bfloat16),
                preferred_element_type=jnp.float32)
        + br_ref[...], 0.0)


def _combine_next(agg_ref, r_ref, b_ref, g_ref, be_ref,
                  w_ref, h_ref, x_ref):
    inv = 1.0 / jnp.sqrt(jnp.float32(1.0 + 1e-5))
    nrows = r_ref.shape[0]
    agg = (agg_ref[0, :nrows, :].astype(jnp.float32)
           + agg_ref[1, :nrows, :].astype(jnp.float32))
    x = ((g_ref[...] * inv)
         * (jnp.maximum(agg + b_ref[...], 0.0) + r_ref[...]) + be_ref[...])
    x_ref[...] = x
    h_ref[...] = jnp.dot(
        x.astype(jnp.bfloat16), w_ref[...].astype(jnp.bfloat16),
        preferred_element_type=jnp.float32).astype(h_ref.dtype)


def _combine_out(agg_ref, r_ref, b_ref, g_ref, be_ref, o_ref):
    inv = 1.0 / jnp.sqrt(jnp.float32(1.0 + 1e-5))
    nrows = r_ref.shape[0]
    agg = (agg_ref[0, :nrows, :].astype(jnp.float32)
           + agg_ref[1, :nrows, :].astype(jnp.float32))
    out = ((g_ref[...] * inv)
           * (jnp.maximum(agg + b_ref[...], 0.0) + r_ref[...])
           + be_ref[...])
    o_ref[...] = out.T


def kernel(feats, edge_index, W1, b1, Wr1, br1, g1, be1,
           W2, b2, Wr2, br2, g2, be2):
    n, d_in = feats.shape
    h = W1.shape[1]
    rpt = ((n + 8 * _NS - 1) // (8 * _NS)) * 8
    sc_dt = jnp.bfloat16
    zeros = jnp.zeros((rpt, h), sc_dt)

    f32 = jnp.float32
    b1r, br1r, g1r, be1r = (v.reshape(1, h) for v in (b1, br1, g1, be1))
    b2r, br2r, g2r, be2r = (v.reshape(1, h) for v in (b2, br2, g2, be2))

    h1 = pl.pallas_call(
        _dense_h,
        out_shape=jax.ShapeDtypeStruct((n, h), sc_dt),
    )(feats, W1)

    agg1 = _sc_segment_sum(h1, edge_index, zeros)

    r1 = pl.pallas_call(
        _dense_res,
        out_shape=jax.ShapeDtypeStruct((n, h), f32),
    )(feats, Wr1, br1r)

    h2, x2 = pl.pallas_call(
        _combine_next,
        out_shape=[jax.ShapeDtypeStruct((n, h), sc_dt),
                   jax.ShapeDtypeStruct((n, h), f32)],
    )(agg1, r1, b1r, g1r, be1r, W2)

    agg2 = _sc_segment_sum(h2, edge_index, zeros)

    r2 = pl.pallas_call(
        _dense_res,
        out_shape=jax.ShapeDtypeStruct((n, h), f32),
    )(x2, Wr2, br2r)

    out_t = pl.pallas_call(
        _combine_out,
        out_shape=jax.ShapeDtypeStruct((h, n), f32),
    )(agg2, r2, b2r, g2r, be2r)

    return out_t.T
